# baseline (speedup 1.0000x reference)
import jax
import jax.numpy as jnp
from jax.experimental import pallas as pl

BATCH = 32
SEQ_LEN = 8192
N_ALLELES = 4
D_MODEL = 64
PACK = 128 // N_ALLELES
R_PER_BATCH = SEQ_LEN // PACK
J = PACK * D_MODEL


def _body(x_ref, w_ref, p_ref, o_ref):
    o_ref[...] = jax.lax.dot_general(
        x_ref[...], w_ref[...],
        dimension_numbers=(((1,), (0,)), ((), ())),
        preferred_element_type=jnp.float32,
    ) + p_ref[...]


def kernel(x, allele_embedding, position_table):
    xv = x.reshape(BATCH * R_PER_BATCH, 128)
    w = jnp.kron(jnp.eye(PACK, dtype=x.dtype), allele_embedding)
    pv = position_table[:SEQ_LEN].reshape(R_PER_BATCH, J)
    out = pl.pallas_call(
        _body,
        grid=(BATCH,),
        in_specs=[
            pl.BlockSpec((R_PER_BATCH, 128), lambda b: (b, 0)),
            pl.BlockSpec((128, J), lambda b: (0, 0)),
            pl.BlockSpec((R_PER_BATCH, J), lambda b: (0, 0)),
        ],
        out_specs=pl.BlockSpec((R_PER_BATCH, J), lambda b: (b, 0)),
        out_shape=jax.ShapeDtypeStruct((BATCH * R_PER_BATCH, J), jnp.float32),
    )(xv, w, pv)
    return out.reshape(BATCH, SEQ_LEN, D_MODEL)


# speedup vs baseline: 1.0725x; 1.0725x over previous
import jax
import jax.numpy as jnp
from jax.experimental import pallas as pl

BATCH = 32
SEQ_LEN = 8192
N_ALLELES = 4
D_MODEL = 64
S_TILE = 2048
S_TILES = SEQ_LEN // S_TILE


def _body(x_ref, a_ref, p_ref, o_ref):
    xb = x_ref[0]
    acc = p_ref[...]
    for n in range(N_ALLELES):
        acc = acc + xb[:, n:n + 1] * a_ref[n:n + 1, :]
    o_ref[0] = acc


def kernel(x, allele_embedding, position_table):
    return pl.pallas_call(
        _body,
        grid=(S_TILES, BATCH),
        in_specs=[
            pl.BlockSpec((1, S_TILE, N_ALLELES), lambda s, b: (b, s, 0)),
            pl.BlockSpec((N_ALLELES, D_MODEL), lambda s, b: (0, 0)),
            pl.BlockSpec((S_TILE, D_MODEL), lambda s, b: (s, 0)),
        ],
        out_specs=pl.BlockSpec((1, S_TILE, D_MODEL), lambda s, b: (b, s, 0)),
        out_shape=jax.ShapeDtypeStruct((BATCH, SEQ_LEN, D_MODEL), jnp.float32),
    )(x, allele_embedding, position_table)
